# Initial kernel scaffold; baseline (speedup 1.0000x reference)
import jax
import jax.numpy as jnp
from jax.experimental import pallas as pl


def kernel(x, edge_index, W1, b1, W2, b2, fc1_w, fc1_b, fc2_w, fc2_b):
    raise NotImplementedError("write your pallas kernel here")


# speedup vs baseline: 125.9248x; 125.9248x over previous
import functools
import numpy as np
import jax
import jax.numpy as jnp
from jax.experimental import pallas as pl
from jax.experimental.pallas import tpu as pltpu


def _round_up(v, m):
    return ((v + m - 1) // m) * m


def _prop_kernel(a_ref, tx1_ref, tx0_ref, w_ref, acc_ref, b_ref,
                 tx2_ref, out_ref, *, first, last):
    t = jnp.dot(a_ref[...], tx1_ref[...], preferred_element_type=jnp.float32)
    if not first:
        t = 2.0 * t - tx0_ref[...]
    tx2_ref[...] = t
    o = acc_ref[...] + jnp.dot(t, w_ref[...], preferred_element_type=jnp.float32)
    if last:
        o = jnp.maximum(o + b_ref[0:1, :], 0.0)
    out_ref[...] = o


def _init_kernel(x_ref, w_ref, o_ref):
    o_ref[...] = jnp.dot(x_ref[...], w_ref[...],
                         preferred_element_type=jnp.float32)


def _fc_kernel(x_ref, w_ref, b_ref, w2_ref, b2_ref, o_ref, acc_ref):
    @pl.when(pl.program_id(0) == 0)
    def _():
        acc_ref[...] = jnp.zeros_like(acc_ref)

    acc_ref[...] += jnp.dot(x_ref[...], w_ref[...],
                            preferred_element_type=jnp.float32)

    @pl.when(pl.program_id(0) == pl.num_programs(0) - 1)
    def _():
        h = acc_ref[...] + b_ref[...]
        z = jnp.dot(h, w2_ref[...], preferred_element_type=jnp.float32)
        z = z + b2_ref[...]
        m = jnp.max(z, axis=1, keepdims=True)
        e = jnp.exp(z - m)
        lse = jnp.log(jnp.sum(e, axis=1, keepdims=True)) + m
        o_ref[...] = z - lse


def _prop(a, tx1, tx0, w, acc, bvec, *, first, last, rb):
    np_, fi = tx1.shape
    fo = w.shape[1]
    kern = functools.partial(_prop_kernel, first=first, last=last)
    return pl.pallas_call(
        kern,
        grid=(np_ // rb,),
        in_specs=[
            pl.BlockSpec((rb, np_), lambda i: (i, 0)),
            pl.BlockSpec((np_, fi), lambda i: (0, 0)),
            pl.BlockSpec((rb, fi), lambda i: (i, 0)),
            pl.BlockSpec((fi, fo), lambda i: (0, 0)),
            pl.BlockSpec((rb, fo), lambda i: (i, 0)),
            pl.BlockSpec((8, fo), lambda i: (0, 0)),
        ],
        out_specs=[
            pl.BlockSpec((rb, fi), lambda i: (i, 0)),
            pl.BlockSpec((rb, fo), lambda i: (i, 0)),
        ],
        out_shape=[
            jax.ShapeDtypeStruct((np_, fi), jnp.float32),
            jax.ShapeDtypeStruct((np_, fo), jnp.float32),
        ],
        input_output_aliases={4: 1},
    )(a, tx1, tx0, w, acc, bvec)


def _cheb_layer(a, x0, wbd, bvec, rb):
    np_, fi = x0.shape
    k_total, _, fo = wbd.shape
    acc = pl.pallas_call(
        _init_kernel,
        grid=(np_ // rb,),
        in_specs=[
            pl.BlockSpec((rb, fi), lambda i: (i, 0)),
            pl.BlockSpec((fi, fo), lambda i: (0, 0)),
        ],
        out_specs=pl.BlockSpec((rb, fo), lambda i: (i, 0)),
        out_shape=jax.ShapeDtypeStruct((np_, fo), jnp.float32),
    )(x0, wbd[0])

    tx2, acc = _prop(a, x0, x0, wbd[1], acc, bvec,
                     first=True, last=False, rb=rb)
    tx0, tx1 = x0, tx2

    def body(k, carry):
        tx0, tx1, acc = carry
        w = jax.lax.dynamic_slice_in_dim(wbd, k, 1, 0)[0]
        tx2, acc2 = _prop(a, tx1, tx0, w, acc, bvec,
                          first=False, last=False, rb=rb)
        return (tx1, tx2, acc2)

    tx0, tx1, acc = jax.lax.fori_loop(2, k_total - 1, body, (tx0, tx1, acc))
    _, out = _prop(a, tx1, tx0, wbd[k_total - 1], acc, bvec,
                   first=False, last=True, rb=rb)
    return out


def kernel(x, edge_index, W1, b1, W2, b2, fc1_w, fc1_b, fc2_w, fc2_b):
    B, N, T = x.shape
    G1 = W1.shape[2]
    G2 = W2.shape[2]
    C = fc1_w.shape[1]
    D = fc2_w.shape[1]

    rb = 256
    np_ = _round_up(N, rb)

    src = edge_index[0].astype(jnp.int32)
    dst = edge_index[1].astype(jnp.int32)
    deg = jnp.zeros((N,), jnp.float32).at[dst].add(1.0)
    dinv = jnp.where(deg > 0, jax.lax.rsqrt(deg), 0.0)
    norm = -dinv[src] * dinv[dst]
    a = jnp.zeros((np_, np_), jnp.float32).at[dst, src].add(norm)

    t_idx = jnp.arange(T, dtype=jnp.float32)
    cosmat = jnp.cos(2.0 * np.pi * jnp.outer(t_idx, t_idx) / T)
    w1c = jnp.einsum('tu,kug->ktg', cosmat, W1)
    eye_b = jnp.eye(B, dtype=jnp.float32)
    wbd1 = jax.vmap(lambda w: jnp.kron(eye_b, w))(w1c)
    wbd2 = jax.vmap(lambda w: jnp.kron(eye_b, w))(W2)

    fi1 = _round_up(B * T, 128)
    wbd1 = jnp.pad(wbd1, ((0, 0), (0, fi1 - B * T), (0, 0)))

    x0 = jnp.transpose(x, (1, 0, 2)).reshape(N, B * T)
    x0 = jnp.pad(x0, ((0, np_ - N), (0, fi1 - B * T)))

    b1v = jnp.tile(jnp.tile(b1, B)[None, :], (8, 1))
    b2v = jnp.tile(jnp.tile(b2, B)[None, :], (8, 1))

    h1 = _cheb_layer(a, x0, wbd1, b1v, rb)
    h2 = _cheb_layer(a, h1, wbd2, b2v, rb)

    hf = jnp.transpose(h2[:N].reshape(N, B, G2), (1, 0, 2)).reshape(B, N * G2)

    dp = 128
    w2p = jnp.pad(fc2_w, ((0, 0), (0, dp - D)))
    b2p = jnp.concatenate([fc2_b, jnp.full((dp - D,), -1e30, jnp.float32)])
    b2p = jnp.tile(b2p[None, :], (B, 1))
    b1p = jnp.tile(fc1_b[None, :], (B, 1))

    kb = 1024
    n_k = (N * G2) // kb
    out = pl.pallas_call(
        _fc_kernel,
        grid=(n_k,),
        in_specs=[
            pl.BlockSpec((B, kb), lambda i: (0, i)),
            pl.BlockSpec((kb, C), lambda i: (i, 0)),
            pl.BlockSpec((B, C), lambda i: (0, 0)),
            pl.BlockSpec((C, dp), lambda i: (0, 0)),
            pl.BlockSpec((B, dp), lambda i: (0, 0)),
        ],
        out_specs=pl.BlockSpec((B, dp), lambda i: (0, 0)),
        out_shape=jax.ShapeDtypeStruct((B, dp), jnp.float32),
        scratch_shapes=[pltpu.VMEM((B, C), jnp.float32)],
    )(hf, fc1_w, b1p, w2p, b2p)
    return out[:, :D]
